# baseline (speedup 1.0000x reference)
import functools

import jax
import jax.numpy as jnp
from jax import lax
from jax.experimental import pallas as pl


def _fold421_lanes(a):
    a = a[:, 0:4] + a[:, 4:8]
    a = a[:, 0:2] + a[:, 2:4]
    return a[:, 0:1] + a[:, 1:2]


def _lane64_reduce(s):
    acc = s[:, 0:8]
    for k in range(1, 8):
        acc = acc + s[:, 8 * k:8 * k + 8]
    return _fold421_lanes(acc)


def _score_body(ids_ref, tok_ref, xs_ref, pos_ref, ker_ref):
    ids = ids_ref[0]
    t = tok_ref[0]
    xs = xs_ref[0]

    mask = (ids >= 0).astype(jnp.float32)
    count = jnp.sum(mask)
    denom = jnp.maximum(count, jnp.float32(1.0))

    parts = []
    for w0 in range(0, 512, 128):
        acc = t[w0:w0 + 8]
        for j in range(w0 + 8, w0 + 128, 8):
            acc = acc + t[j:j + 8]
        a = acc[0:4] + acc[4:8]
        a = a[0:2] + a[2:4]
        parts.append(a[0:1] + a[1:2])
    ksum = parts[0]
    for p in parts[1:]:
        ksum = ksum + p
    kv = ksum / denom
    ker_ref[0] = kv

    kn2 = _lane64_reduce(kv * kv)
    k_norm = jnp.maximum(jnp.sqrt(kn2), jnp.float32(1e-8))

    xs_n2 = _lane64_reduce(xs * xs)
    xs_norm = jnp.maximum(jnp.sqrt(xs_n2), jnp.float32(1e-8))

    kpad = jnp.concatenate([kv, jnp.zeros((7, 64), jnp.float32)], axis=0)
    rhs = kpad.astype(jnp.bfloat16).T
    dg = lax.dot_general(xs.astype(jnp.bfloat16), rhs,
                         (((1,), (0,)), ((), ())),
                         preferred_element_type=jnp.float32)
    dots = dg[:, 0:1]

    cos = dots / (xs_norm * k_norm)
    pos = (cos + jnp.float32(1.0)) / jnp.float32(2.0)
    gate = (count > 0).astype(jnp.float32)
    pos_ref[0] = (pos * gate).reshape(1, 2048)


def _topk_body(pos_ref, val_ref, idx_ref):
    v0 = pos_ref[...]
    iota = lax.broadcasted_iota(jnp.int32, (16, 2048), 1)
    riota = lax.broadcasted_iota(jnp.int32, (16, 256), 1)

    def body(r, carry):
        v, vals, idxs = carry
        mx = jnp.max(v, axis=1, keepdims=True)
        cand = jnp.where(v == mx, iota, jnp.int32(2048))
        am = jnp.min(cand, axis=1, keepdims=True)
        sel = riota == r
        vals = jnp.where(sel, mx, vals)
        idxs = jnp.where(sel, am, idxs)
        v = jnp.where(iota == am, -jnp.inf, v)
        return v, vals, idxs

    vals0 = jnp.zeros((16, 256), jnp.float32)
    idxs0 = jnp.zeros((16, 256), jnp.int32)
    _, vals, idxs = lax.fori_loop(0, 256, body, (v0, vals0, idxs0))
    val_ref[...] = vals
    idx_ref[...] = idxs


def _onehot_body(idx_ref, val_ref, out_ref):
    idr = idx_ref[0, 0]
    valr = val_ref[0, 0]
    idc = idr.reshape(64, 1)
    vc = valr.reshape(64, 1)
    iota = lax.broadcasted_iota(jnp.int32, (64, 2048), 1)
    oh = (iota == idc) & (vc > 0)
    out_ref[0] = oh.astype(jnp.float32)


def kernel(x_b, x_s, base_idxs):
    B, N, C = x_b.shape
    m = base_idxs.shape[1] // 2
    k = N // 8

    ids = base_idxs[:, :m]
    mask = ids >= 0
    idsc = jnp.where(mask, ids, 0)
    tokens = jnp.take_along_axis(x_b, idsc[:, :, None], axis=1)
    tokens = tokens * mask[:, :, None].astype(x_b.dtype)
    ids3 = ids.reshape(B, 1, m)

    pos3, ker3 = pl.pallas_call(
        _score_body,
        grid=(B,),
        in_specs=[
            pl.BlockSpec((1, 1, m), lambda b: (b, 0, 0)),
            pl.BlockSpec((1, m, C), lambda b: (b, 0, 0)),
            pl.BlockSpec((1, N, C), lambda b: (b, 0, 0)),
        ],
        out_specs=[
            pl.BlockSpec((1, 1, N), lambda b: (b, 0, 0)),
            pl.BlockSpec((1, 1, C), lambda b: (b, 0, 0)),
        ],
        out_shape=[
            jax.ShapeDtypeStruct((B, 1, N), jnp.float32),
            jax.ShapeDtypeStruct((B, 1, C), jnp.float32),
        ],
    )(ids3, tokens, x_s)
    pos_scores = pos3.reshape(B, N)
    kernels = ker3.reshape(B, C)

    topk_val, topk_idx = pl.pallas_call(
        _topk_body,
        out_shape=[
            jax.ShapeDtypeStruct((B, k), jnp.float32),
            jax.ShapeDtypeStruct((B, k), jnp.int32),
        ],
    )(pos_scores)

    idx4 = topk_idx.reshape(B, 4, 1, 64)
    val4 = topk_val.reshape(B, 4, 1, 64)
    selected = pl.pallas_call(
        _onehot_body,
        grid=(B, 4),
        in_specs=[
            pl.BlockSpec((1, 1, 1, 64), lambda b, j: (b, j, 0, 0)),
            pl.BlockSpec((1, 1, 1, 64), lambda b, j: (b, j, 0, 0)),
        ],
        out_specs=pl.BlockSpec((1, 64, N), lambda b, j: (b, j, 0)),
        out_shape=jax.ShapeDtypeStruct((B, k, N), jnp.float32),
    )(idx4, val4)

    return (selected, topk_idx, pos_scores, x_s, kernels[:, :, None])


# speedup vs baseline: 1.1700x; 1.1700x over previous
import functools

import jax
import jax.numpy as jnp
from jax import lax
from jax.experimental import pallas as pl


def _fold421_lanes(a):
    a = a[:, 0:4] + a[:, 4:8]
    a = a[:, 0:2] + a[:, 2:4]
    return a[:, 0:1] + a[:, 1:2]


def _lane64_reduce(s):
    acc = s[:, 0:8]
    for k in range(1, 8):
        acc = acc + s[:, 8 * k:8 * k + 8]
    return _fold421_lanes(acc)


def _score_body(ids_ref, tok_ref, xs_ref, pos_ref, ker_ref):
    ids = ids_ref[0]
    t = tok_ref[0]
    xs = xs_ref[0]

    mask = (ids >= 0).astype(jnp.float32)
    count = jnp.sum(mask)
    denom = jnp.maximum(count, jnp.float32(1.0))

    parts = []
    for w0 in range(0, 512, 128):
        acc = t[w0:w0 + 8]
        for j in range(w0 + 8, w0 + 128, 8):
            acc = acc + t[j:j + 8]
        a = acc[0:4] + acc[4:8]
        a = a[0:2] + a[2:4]
        parts.append(a[0:1] + a[1:2])
    ksum = parts[0]
    for p in parts[1:]:
        ksum = ksum + p
    kv = ksum / denom
    ker_ref[0] = kv

    kn2 = _lane64_reduce(kv * kv)
    k_norm = jnp.maximum(jnp.sqrt(kn2), jnp.float32(1e-8))

    xst = xs.T
    sq = xst * xst
    accn = sq[0:8]
    for kk in range(1, 8):
        accn = accn + sq[8 * kk:8 * kk + 8]
    a4 = accn[0:4] + accn[4:8]
    a2 = a4[0:2] + a4[2:4]
    xs_n2 = a2[0:1] + a2[1:2]
    xs_norm = jnp.maximum(jnp.sqrt(xs_n2), jnp.float32(1e-8))

    kpad = jnp.concatenate([kv, jnp.zeros((7, 64), jnp.float32)], axis=0)
    dg = lax.dot_general(kpad.astype(jnp.bfloat16), xst.astype(jnp.bfloat16),
                         (((1,), (0,)), ((), ())),
                         preferred_element_type=jnp.float32)
    dots = dg[0:1, :]

    cos = dots / (xs_norm * k_norm)
    pos = (cos + jnp.float32(1.0)) / jnp.float32(2.0)
    gate = (count > 0).astype(jnp.float32)
    pos_ref[0] = pos * gate


def _topk_body(pos_ref, val_ref, idx_ref):
    v0 = pos_ref[...]
    iota = lax.broadcasted_iota(jnp.int32, (16, 2048), 1)
    riota = lax.broadcasted_iota(jnp.int32, (16, 256), 1)

    def _rowmax(v):
        g = v.reshape(16, 16, 128)
        t = jnp.max(g, axis=1)
        return jnp.max(t, axis=1, keepdims=True)

    def _rowmin(v):
        g = v.reshape(16, 16, 128)
        t = jnp.min(g, axis=1)
        return jnp.min(t, axis=1, keepdims=True)

    def body(r, carry):
        v, vals, idxs = carry
        mx = _rowmax(v)
        cand = jnp.where(v == mx, iota, jnp.int32(2048))
        am = _rowmin(cand)
        sel = riota == r
        vals = jnp.where(sel, mx, vals)
        idxs = jnp.where(sel, am, idxs)
        v = jnp.where(iota == am, -jnp.inf, v)
        return v, vals, idxs

    vals0 = jnp.zeros((16, 256), jnp.float32)
    idxs0 = jnp.zeros((16, 256), jnp.int32)
    _, vals, idxs = lax.fori_loop(0, 256, body, (v0, vals0, idxs0))
    val_ref[...] = vals
    idx_ref[...] = idxs


def _onehot_body(idx_ref, val_ref, out_ref):
    idr = idx_ref[0, 0]
    valr = val_ref[0, 0]
    idc = idr.reshape(64, 1)
    vc = valr.reshape(64, 1)
    iota = lax.broadcasted_iota(jnp.int32, (64, 2048), 1)
    oh = (iota == idc) & (vc > 0)
    out_ref[0] = oh.astype(jnp.float32)


def kernel(x_b, x_s, base_idxs):
    B, N, C = x_b.shape
    m = base_idxs.shape[1] // 2
    k = N // 8

    ids = base_idxs[:, :m]
    mask = ids >= 0
    idsc = jnp.where(mask, ids, 0)
    tokens = jnp.take_along_axis(x_b, idsc[:, :, None], axis=1)
    tokens = tokens * mask[:, :, None].astype(x_b.dtype)
    ids3 = ids.reshape(B, 1, m)

    pos3, ker3 = pl.pallas_call(
        _score_body,
        grid=(B,),
        in_specs=[
            pl.BlockSpec((1, 1, m), lambda b: (b, 0, 0)),
            pl.BlockSpec((1, m, C), lambda b: (b, 0, 0)),
            pl.BlockSpec((1, N, C), lambda b: (b, 0, 0)),
        ],
        out_specs=[
            pl.BlockSpec((1, 1, N), lambda b: (b, 0, 0)),
            pl.BlockSpec((1, 1, C), lambda b: (b, 0, 0)),
        ],
        out_shape=[
            jax.ShapeDtypeStruct((B, 1, N), jnp.float32),
            jax.ShapeDtypeStruct((B, 1, C), jnp.float32),
        ],
    )(ids3, tokens, x_s)
    pos_scores = pos3.reshape(B, N)
    kernels = ker3.reshape(B, C)

    topk_val, topk_idx = pl.pallas_call(
        _topk_body,
        out_shape=[
            jax.ShapeDtypeStruct((B, k), jnp.float32),
            jax.ShapeDtypeStruct((B, k), jnp.int32),
        ],
    )(pos_scores)

    idx4 = topk_idx.reshape(B, 4, 1, 64)
    val4 = topk_val.reshape(B, 4, 1, 64)
    selected = pl.pallas_call(
        _onehot_body,
        grid=(B, 4),
        in_specs=[
            pl.BlockSpec((1, 1, 1, 64), lambda b, j: (b, j, 0, 0)),
            pl.BlockSpec((1, 1, 1, 64), lambda b, j: (b, j, 0, 0)),
        ],
        out_specs=pl.BlockSpec((1, 64, N), lambda b, j: (b, j, 0)),
        out_shape=jax.ShapeDtypeStruct((B, k, N), jnp.float32),
    )(idx4, val4)

    return (selected, topk_idx, pos_scores, x_s, kernels[:, :, None])


# speedup vs baseline: 1.2693x; 1.0849x over previous
import functools

import jax
import jax.numpy as jnp
from jax import lax
from jax.experimental import pallas as pl


def _fold421_lanes(a):
    a = a[:, 0:4] + a[:, 4:8]
    a = a[:, 0:2] + a[:, 2:4]
    return a[:, 0:1] + a[:, 1:2]


def _lane64_reduce(s):
    acc = s[:, 0:8]
    for k in range(1, 8):
        acc = acc + s[:, 8 * k:8 * k + 8]
    return _fold421_lanes(acc)


def _score_body(ids_ref, tok_ref, xs_ref, pos_ref, ker_ref):
    ids = ids_ref[0]
    t = tok_ref[0]
    xs = xs_ref[0]

    mask = (ids >= 0).astype(jnp.float32)
    count = jnp.sum(mask)
    denom = jnp.maximum(count, jnp.float32(1.0))

    parts = []
    for w0 in range(0, 512, 128):
        acc = t[w0:w0 + 8]
        for j in range(w0 + 8, w0 + 128, 8):
            acc = acc + t[j:j + 8]
        a = acc[0:4] + acc[4:8]
        a = a[0:2] + a[2:4]
        parts.append(a[0:1] + a[1:2])
    ksum = parts[0]
    for p in parts[1:]:
        ksum = ksum + p
    kv = ksum / denom
    ker_ref[0] = kv

    kn2 = _lane64_reduce(kv * kv)
    k_norm = jnp.maximum(jnp.sqrt(kn2), jnp.float32(1e-8))

    xst = xs.T
    sq = xst * xst
    accn = sq[0:8]
    for kk in range(1, 8):
        accn = accn + sq[8 * kk:8 * kk + 8]
    a4 = accn[0:4] + accn[4:8]
    a2 = a4[0:2] + a4[2:4]
    xs_n2 = a2[0:1] + a2[1:2]
    xs_norm = jnp.maximum(jnp.sqrt(xs_n2), jnp.float32(1e-8))

    kpad = jnp.concatenate([kv, jnp.zeros((7, 64), jnp.float32)], axis=0)
    dg = lax.dot_general(kpad.astype(jnp.bfloat16), xst.astype(jnp.bfloat16),
                         (((1,), (0,)), ((), ())),
                         preferred_element_type=jnp.float32)
    dots = dg[0:1, :]

    cos = dots / (xs_norm * k_norm)
    pos = (cos + jnp.float32(1.0)) / jnp.float32(2.0)
    gate = (count > 0).astype(jnp.float32)
    pos_ref[0] = pos * gate


def _topk_body(pos_ref, val_ref, idx_ref):
    v0 = pos_ref[...]
    iota = lax.broadcasted_iota(jnp.int32, (16, 2048), 1)
    riota = lax.broadcasted_iota(jnp.int32, (16, 256), 1)

    def _rowmax(v):
        ts = [v[:, 128 * g:128 * (g + 1)] for g in range(16)]
        while len(ts) > 1:
            ts = [jnp.maximum(ts[2 * i], ts[2 * i + 1]) for i in range(len(ts) // 2)]
        return jnp.max(ts[0], axis=1, keepdims=True)

    def _rowmin(v):
        ts = [v[:, 128 * g:128 * (g + 1)] for g in range(16)]
        while len(ts) > 1:
            ts = [jnp.minimum(ts[2 * i], ts[2 * i + 1]) for i in range(len(ts) // 2)]
        return jnp.min(ts[0], axis=1, keepdims=True)

    def body(r, carry):
        v, vals, idxs = carry
        mx = _rowmax(v)
        cand = jnp.where(v == mx, iota, jnp.int32(2048))
        am = _rowmin(cand)
        sel = riota == r
        vals = jnp.where(sel, mx, vals)
        idxs = jnp.where(sel, am, idxs)
        v = jnp.where(iota == am, -jnp.inf, v)
        return v, vals, idxs

    vals0 = jnp.zeros((16, 256), jnp.float32)
    idxs0 = jnp.zeros((16, 256), jnp.int32)
    _, vals, idxs = lax.fori_loop(0, 256, body, (v0, vals0, idxs0))
    val_ref[...] = vals
    idx_ref[...] = idxs


def _onehot_body(idx_ref, val_ref, out_ref):
    idr = idx_ref[0, 0]
    valr = val_ref[0, 0]
    idc = idr.reshape(64, 1)
    vc = valr.reshape(64, 1)
    iota = lax.broadcasted_iota(jnp.int32, (64, 2048), 1)
    oh = (iota == idc) & (vc > 0)
    out_ref[0] = oh.astype(jnp.float32)


def kernel(x_b, x_s, base_idxs):
    B, N, C = x_b.shape
    m = base_idxs.shape[1] // 2
    k = N // 8

    ids = base_idxs[:, :m]
    mask = ids >= 0
    idsc = jnp.where(mask, ids, 0)
    tokens = jnp.take_along_axis(x_b, idsc[:, :, None], axis=1)
    tokens = tokens * mask[:, :, None].astype(x_b.dtype)
    ids3 = ids.reshape(B, 1, m)

    pos3, ker3 = pl.pallas_call(
        _score_body,
        grid=(B,),
        in_specs=[
            pl.BlockSpec((1, 1, m), lambda b: (b, 0, 0)),
            pl.BlockSpec((1, m, C), lambda b: (b, 0, 0)),
            pl.BlockSpec((1, N, C), lambda b: (b, 0, 0)),
        ],
        out_specs=[
            pl.BlockSpec((1, 1, N), lambda b: (b, 0, 0)),
            pl.BlockSpec((1, 1, C), lambda b: (b, 0, 0)),
        ],
        out_shape=[
            jax.ShapeDtypeStruct((B, 1, N), jnp.float32),
            jax.ShapeDtypeStruct((B, 1, C), jnp.float32),
        ],
    )(ids3, tokens, x_s)
    pos_scores = pos3.reshape(B, N)
    kernels = ker3.reshape(B, C)

    topk_val, topk_idx = pl.pallas_call(
        _topk_body,
        out_shape=[
            jax.ShapeDtypeStruct((B, k), jnp.float32),
            jax.ShapeDtypeStruct((B, k), jnp.int32),
        ],
    )(pos_scores)

    idx4 = topk_idx.reshape(B, 4, 1, 64)
    val4 = topk_val.reshape(B, 4, 1, 64)
    selected = pl.pallas_call(
        _onehot_body,
        grid=(B, 4),
        in_specs=[
            pl.BlockSpec((1, 1, 1, 64), lambda b, j: (b, j, 0, 0)),
            pl.BlockSpec((1, 1, 1, 64), lambda b, j: (b, j, 0, 0)),
        ],
        out_specs=pl.BlockSpec((1, 64, N), lambda b, j: (b, j, 0)),
        out_shape=jax.ShapeDtypeStruct((B, k, N), jnp.float32),
    )(idx4, val4)

    return (selected, topk_idx, pos_scores, x_s, kernels[:, :, None])
